# Initial kernel scaffold; baseline (speedup 1.0000x reference)
import jax
import jax.numpy as jnp
from jax.experimental import pallas as pl


def kernel(x, table):
    raise NotImplementedError("write your pallas kernel here")


# speedup vs baseline: 1.9739x; 1.9739x over previous
import jax
import jax.numpy as jnp
from jax.experimental import pallas as pl


def _body(x_ref, t_ref, o_ref):
    o_ref[...] = x_ref[...] + jnp.transpose(t_ref[...])[None]


def kernel(x, table):
    B, C, T = x.shape
    Tb = 512
    return pl.pallas_call(
        _body,
        grid=(T // Tb,),
        in_specs=[
            pl.BlockSpec((B, C, Tb), lambda i: (0, 0, i)),
            pl.BlockSpec((Tb, C), lambda i: (i, 0)),
        ],
        out_specs=pl.BlockSpec((B, C, Tb), lambda i: (0, 0, i)),
        out_shape=jax.ShapeDtypeStruct((B, C, T), x.dtype),
    )(x, table)
